# baseline (speedup 1.0000x reference)
import functools

import jax
import jax.numpy as jnp
from jax import lax
from jax.experimental import pallas as pl
from jax.experimental.pallas import tpu as pltpu
from jax.experimental.pallas import tpu_sc as plsc

HID = 128
KNN = 16
NSEQ = 4
NSLOT = NSEQ + KNN
STEP = 0.1
N = 2048
B = 4
RB = 256
OFFS = (-2, -1, 1, 2)


def _silu(v):
    return v * jax.nn.sigmoid(v)


def _knn_body(xpad_ref, xt_ref, h_ref, w1a_ref, w1b_ref, b1_ref,
              nn_ref, wk_ref, a_ref, bb_ref):
    xr = xpad_ref[0]
    xt = xt_ref[0]
    sqj = jnp.sum(xt * xt, axis=0, keepdims=True)
    rb = pl.program_id(1)
    maxi = jnp.int32(0x7FFFFFFF)
    SLAB = 16
    nn_slabs = []
    for sl in range(RB // SLAB):
        xs = xr[sl * SLAB:(sl + 1) * SLAB]
        mm = jax.lax.dot_general(xs, xt, (((1,), (0,)), ((), ())),
                                 preferred_element_type=jnp.float32)
        sqi = jnp.sum(xs * xs, axis=1, keepdims=True)
        d2 = sqi + sqj - 2.0 * mm
        row0 = rb * RB + sl * SLAB
        rows = row0 + jax.lax.broadcasted_iota(jnp.int32, (SLAB, N), 0)
        cols = jax.lax.broadcasted_iota(jnp.int32, (SLAB, N), 1)
        d2 = jnp.where(rows == cols, jnp.inf, d2)
        bits = jax.lax.bitcast_convert_type(d2, jnp.int32)
        key = bits ^ ((bits >> 31) & jnp.int32(0x7FFFFFFF))
        kp = (key & jnp.int32(~0x7FF)) | cols
        q0, q1, q2, q3 = (kp[:, :512], kp[:, 512:1024],
                          kp[:, 1024:1536], kp[:, 1536:])
        a0, a1 = jnp.minimum(q0, q1), jnp.maximum(q0, q1)
        b0, b1 = jnp.minimum(q2, q3), jnp.maximum(q2, q3)
        s0, s3 = jnp.minimum(a0, b0), jnp.maximum(a1, b1)
        mid1, mid2 = jnp.maximum(a0, b0), jnp.minimum(a1, b1)
        s1, s2 = jnp.minimum(mid1, mid2), jnp.maximum(mid1, mid2)
        pos = jax.lax.broadcasted_iota(jnp.int32, (SLAB, 512), 1)
        picks = []
        for _ in range(KNN):
            m = jnp.min(s0, axis=1, keepdims=True)
            picks.append(m & jnp.int32(0x7FF))
            cond = pos == (m & jnp.int32(0x1FF))
            s0 = jnp.where(cond, s1, s0)
            s1 = jnp.where(cond, s2, s1)
            s2 = jnp.where(cond, s3, s2)
            s3 = jnp.where(cond, maxi, s3)
        nn_slabs.append(jnp.concatenate(picks, axis=1))
    nn = jnp.concatenate(nn_slabs, axis=0)
    nn_ref[0] = nn
    r16 = rb * RB + jax.lax.broadcasted_iota(jnp.int32, (RB, KNN), 0)
    dup = jnp.zeros((RB, KNN), dtype=jnp.bool_)
    for off in OFFS:
        tgt = r16 + off
        dup = dup | ((nn == tgt) & (tgt >= 0) & (tgt < N))
    wk_ref[0] = 1.0 - dup.astype(jnp.float32)
    h = h_ref[0]
    a_ref[0] = jax.lax.dot_general(h, w1a_ref[...], (((1,), (0,)), ((), ())),
                                   preferred_element_type=jnp.float32) + b1_ref[...]
    bb_ref[0] = jax.lax.dot_general(h, w1b_ref[...], (((1,), (0,)), ((), ())),
                                    preferred_element_type=jnp.float32)


def _run_knn(xpad, xt, hidden, w1a, w1b, b1, interpret=False):
    grid = (B, N // RB)
    return pl.pallas_call(
        _knn_body,
        grid=grid,
        in_specs=[
            pl.BlockSpec((1, RB, 16), lambda b, r: (b, r, 0)),
            pl.BlockSpec((1, 16, N), lambda b, r: (b, 0, 0)),
            pl.BlockSpec((1, RB, HID), lambda b, r: (b, r, 0)),
            pl.BlockSpec((HID, HID), lambda b, r: (0, 0)),
            pl.BlockSpec((HID, HID), lambda b, r: (0, 0)),
            pl.BlockSpec((1, HID), lambda b, r: (0, 0)),
        ],
        out_specs=[
            pl.BlockSpec((1, RB, KNN), lambda b, r: (b, r, 0)),
            pl.BlockSpec((1, RB, KNN), lambda b, r: (b, r, 0)),
            pl.BlockSpec((1, RB, HID), lambda b, r: (b, r, 0)),
            pl.BlockSpec((1, RB, HID), lambda b, r: (b, r, 0)),
        ],
        out_shape=[
            jax.ShapeDtypeStruct((B, N, KNN), jnp.int32),
            jax.ShapeDtypeStruct((B, N, KNN), jnp.float32),
            jax.ShapeDtypeStruct((B, N, HID), jnp.float32),
            jax.ShapeDtypeStruct((B, N, HID), jnp.float32),
        ],
        interpret=interpret,
    )(xpad, xt, hidden, w1a, w1b, b1)


def _egnn_body(h_ref, xpad_ref, a_ref, wk_ref, bg_ref, xg_ref,
               w2_ref, b2_ref, cw1_ref, cb1_ref, cw2_ref, cb2_ref,
               nw1a_ref, nw1b_ref, nb1_ref, nw2_ref, nb2_ref,
               lng_ref, lnb_ref, wlast_ref,
               ho_ref, xo_ref, aggm, aggd, deg):
    k = pl.program_id(0)

    @pl.when(k == 0)
    def _():
        aggm[...] = jnp.zeros_like(aggm)
        aggd[...] = jnp.zeros_like(aggd)
        deg[...] = jnp.zeros_like(deg)

    xi = xpad_ref[...]
    xg = xg_ref[0]
    lane16 = jax.lax.broadcasted_iota(jnp.int32, (N, 16), 1)
    rel = jnp.where(lane16 < 3, xi - xg, 0.0)
    dist2 = jnp.sum(rel * rel, axis=1, keepdims=True)

    icol = jax.lax.broadcasted_iota(jnp.int32, (N, 1), 0)
    off = jnp.where(k == 0, -2, jnp.where(k == 1, -1, jnp.where(k == 2, 1, 2)))
    tgt = icol + off
    wseq = ((tgt >= 0) & (tgt < N)).astype(jnp.float32)
    lane = jax.lax.broadcasted_iota(jnp.int32, (N, KNN), 1)
    wknn = jnp.sum(wk_ref[...] * (lane == (k - NSEQ)).astype(jnp.float32),
                   axis=1, keepdims=True)
    w = jnp.where(k < NSEQ, wseq, wknn)

    pre1 = a_ref[...] + bg_ref[0] + dist2 * wlast_ref[...]
    t = _silu(pre1)
    msg = _silu(jax.lax.dot_general(t.astype(jnp.bfloat16), w2_ref[...],
                                    (((1,), (0,)), ((), ())),
                                    preferred_element_type=jnp.float32) + b2_ref[...])
    c1 = _silu(jax.lax.dot_general(msg.astype(jnp.bfloat16), cw1_ref[...],
                                   (((1,), (0,)), ((), ())),
                                   preferred_element_type=jnp.float32) + cb1_ref[...])
    coef = jnp.tanh(jnp.sum(c1 * cw2_ref[...], axis=1, keepdims=True) + cb2_ref[0, 0])

    aggm[...] += msg * w
    aggd[...] += rel * (coef * w)
    deg[...] += w

    @pl.when(k == NSLOT - 1)
    def _():
        h = h_ref[...]
        hm1 = (jax.lax.dot_general(h.astype(jnp.bfloat16), nw1a_ref[...],
                                   (((1,), (0,)), ((), ())),
                                   preferred_element_type=jnp.float32)
               + jax.lax.dot_general(aggm[...].astype(jnp.bfloat16), nw1b_ref[...],
                                     (((1,), (0,)), ((), ())),
                                     preferred_element_type=jnp.float32)
               + nb1_ref[...])
        hn = h + jax.lax.dot_general(_silu(hm1).astype(jnp.bfloat16), nw2_ref[...],
                                     (((1,), (0,)), ((), ())),
                                     preferred_element_type=jnp.float32) + nb2_ref[...]
        mu = jnp.mean(hn, axis=1, keepdims=True)
        var = jnp.mean((hn - mu) ** 2, axis=1, keepdims=True)
        ho_ref[...] = (hn - mu) / jnp.sqrt(var + 1e-5) * lng_ref[...] + lnb_ref[...]
        xo_ref[...] = xi + STEP * aggd[...] / jnp.maximum(deg[...], 1.0)


def _run_egnn(hidden, xpad, A, wk, bg, xg, p, interpret=False):
    grid = (NSLOT,)
    cvec = lambda v: v.reshape(1, -1)
    bf = jnp.bfloat16
    w2, b2 = p['edge_w2'].astype(bf), cvec(p['edge_b2'])
    cw1, cb1 = p['coord_w1'].astype(bf), cvec(p['coord_b1'])
    cw2 = p['coord_w2'].reshape(1, HID)
    cb2 = p['coord_b2'].reshape(1, 1)
    nw1a = p['node_w1'][:HID].astype(bf)
    nw1b = p['node_w1'][HID:].astype(bf)
    nb1 = cvec(p['node_b1'])
    nw2, nb2 = p['node_w2'].astype(bf), cvec(p['node_b2'])
    lng, lnb = cvec(p['ln_g']), cvec(p['ln_b'])
    wlast = p['edge_w1'][2 * HID].reshape(1, HID)

    full = lambda shp: pl.BlockSpec(shp, lambda k: tuple(0 for _ in shp))
    perk = lambda shp: pl.BlockSpec((1,) + shp, lambda k: (k, 0, 0))

    return pl.pallas_call(
        _egnn_body,
        grid=grid,
        in_specs=[
            full((N, HID)),
            full((N, 16)),
            full((N, HID)),
            full((N, KNN)),
            perk((N, HID)),
            perk((N, 16)),
            full((HID, HID)), full((1, HID)),
            full((HID, HID)), full((1, HID)),
            full((1, HID)), full((1, 1)),
            full((HID, HID)), full((HID, HID)), full((1, HID)),
            full((HID, HID)), full((1, HID)),
            full((1, HID)), full((1, HID)),
            full((1, HID)),
        ],
        out_specs=[
            full((N, HID)),
            full((N, 16)),
        ],
        out_shape=[
            jax.ShapeDtypeStruct((N, HID), jnp.float32),
            jax.ShapeDtypeStruct((N, 16), jnp.float32),
        ],
        scratch_shapes=[
            pltpu.VMEM((N, HID), jnp.float32),
            pltpu.VMEM((N, 16), jnp.float32),
            pltpu.VMEM((N, 1), jnp.float32),
        ],
        interpret=interpret,
    )(hidden, xpad, A, wk, bg, xg,
      w2, b2, cw1, cb1, cw2, cb2, nw1a, nw1b, nb1, nw2, nb2, lng, lnb, wlast)


_NC, _NS = 2, 16
_NW = _NC * _NS
_E = NSLOT * N
_PW = _E // _NW
_CH = 128
_NCHUNK = _PW // _CH


def _gather_sc_body(btab_hbm, xflat_hbm, idx_hbm, bg_hbm, xg_hbm,
                    idx_v, xvm, b0, b1, x0, x1, s0, s1):
    wid = lax.axis_index("s") * _NC + lax.axis_index("c")
    base = wid * _PW
    pltpu.sync_copy(idx_hbm.at[pl.ds(base, _PW)], idx_v)
    pltpu.sync_copy(xflat_hbm, xvm)
    bufs_b, bufs_x, sems = (b0, b1), (x0, x1), (s0, s1)
    lane = jax.lax.iota(jnp.int32, 16)

    def start(j, slot):
        idxs = idx_v.at[pl.ds(j * _CH, _CH)]
        pltpu.async_copy(btab_hbm.at[idxs], bufs_b[slot], sems[slot])

    def gather_x(j, slot):
        xb = bufs_x[slot]
        for g in range(_CH // 16):
            rows = idx_v[pl.ds(j * _CH + g * 16, 16)]
            out_base = (g * 16 + lane) * 16
            for c in range(3):
                v = plsc.load_gather(xvm, [rows * 4 + c])
                plsc.store_scatter(xb, [out_base + c], v)

    def wait(slot):
        pltpu.make_async_copy(btab_hbm.at[pl.ds(0, _CH)], bufs_b[slot], sems[slot]).wait()

    def write(j, slot):
        pltpu.sync_copy(bufs_b[slot], bg_hbm.at[pl.ds(base + j * _CH, _CH)])
        pltpu.sync_copy(bufs_x[slot], xg_hbm.at[pl.ds((base + j * _CH) * 16, _CH * 16)])

    start(0, 0)

    @pl.loop(0, _NCHUNK, step=2)
    def _chunks(jj):
        for t in range(2):
            j = jj + t

            @pl.when(j + 1 < _NCHUNK)
            def _():
                start(j + 1, 1 - t)

            gather_x(j, t)
            wait(t)
            write(j, t)


def _run_gather(btab, xflat, gidx):
    mesh = plsc.VectorSubcoreMesh(core_axis_name="c", subcore_axis_name="s")
    f = pl.kernel(
        _gather_sc_body,
        out_type=[
            jax.ShapeDtypeStruct((_E, HID), jnp.float32),
            jax.ShapeDtypeStruct((_E * 16,), jnp.float32),
        ],
        mesh=mesh,
        scratch_types=[
            pltpu.VMEM((_PW,), jnp.int32),
            pltpu.VMEM((N * 4,), jnp.float32),
            pltpu.VMEM((_CH, HID), jnp.float32),
            pltpu.VMEM((_CH, HID), jnp.float32),
            pltpu.VMEM((_CH * 16,), jnp.float32),
            pltpu.VMEM((_CH * 16,), jnp.float32),
            pltpu.SemaphoreType.DMA,
            pltpu.SemaphoreType.DMA,
        ],
        compiler_params=pltpu.CompilerParams(needs_layout_passes=False),
    )
    return f(btab, xflat, gidx)


def _assemble(hidden, coords, params, interpret=False):
    xpad = jnp.pad(coords, ((0, 0), (0, 0), (0, 13)))
    xt = jnp.transpose(xpad, (0, 2, 1))
    w1a = params['edge_w1'][:HID]
    w1b = params['edge_w1'][HID:2 * HID]
    b1 = params['edge_b1'].reshape(1, HID)

    nn, wk, A, Btab = _run_knn(xpad, xt, hidden, w1a, w1b, b1, interpret)

    idx = jnp.arange(N, dtype=jnp.int32)
    seq = jnp.stack([jnp.clip(idx + o, 0, N - 1) for o in OFFS], axis=1)
    seq = jnp.broadcast_to(seq[None], (B, N, NSEQ))
    nbr = jnp.concatenate([seq, nn], axis=2)
    gidx = jnp.transpose(nbr, (0, 2, 1)).reshape(B, -1)
    xflat = jnp.pad(coords, ((0, 0), (0, 0), (0, 1))).reshape(B, -1)

    hos, xos = [], []
    for b in range(B):
        if interpret:
            bg = jnp.take(Btab[b], gidx[b], axis=0)
            xg = jnp.take(xpad[b], gidx[b], axis=0).reshape(-1)
        else:
            bg, xg = _run_gather(Btab[b], xflat[b], gidx[b])
        bg = bg.reshape(NSLOT, N, HID)
        xg = xg.reshape(NSLOT, N, 16)
        ho, xo = _run_egnn(hidden[b], xpad[b], A[b], wk[b], bg, xg, params,
                           interpret)
        hos.append(ho)
        xos.append(xo[:, :3])
    return jnp.stack(hos), jnp.stack(xos)


def kernel(hidden, coords, mask, params):
    ho, xo = _assemble(hidden, coords, params)
    return (ho, xo)


# speedup vs baseline: 1.0558x; 1.0558x over previous
import functools

import jax
import jax.numpy as jnp
from jax import lax
from jax.experimental import pallas as pl
from jax.experimental.pallas import tpu as pltpu
from jax.experimental.pallas import tpu_sc as plsc

HID = 128
KNN = 16
NSEQ = 4
NSLOT = NSEQ + KNN
STEP = 0.1
N = 2048
B = 4
RB = 256
OFFS = (-2, -1, 1, 2)


def _silu(v):
    return v * jax.nn.sigmoid(v)


def _knn_body(xpad_ref, xt_ref, h_ref, w1a_ref, w1b_ref, b1_ref,
              nn_ref, wk_ref, a_ref, bb_ref):
    xr = xpad_ref[0]
    xt = xt_ref[0]
    sqj = jnp.sum(xt * xt, axis=0, keepdims=True)
    rb = pl.program_id(1)
    maxi = jnp.int32(0x7FFFFFFF)
    SLAB = 16
    nn_slabs = []
    for sl in range(RB // SLAB):
        xs = xr[sl * SLAB:(sl + 1) * SLAB]
        mm = jax.lax.dot_general(xs, xt, (((1,), (0,)), ((), ())),
                                 preferred_element_type=jnp.float32)
        sqi = jnp.sum(xs * xs, axis=1, keepdims=True)
        d2 = sqi + sqj - 2.0 * mm
        row0 = rb * RB + sl * SLAB
        rows = row0 + jax.lax.broadcasted_iota(jnp.int32, (SLAB, N), 0)
        cols = jax.lax.broadcasted_iota(jnp.int32, (SLAB, N), 1)
        d2 = jnp.where(rows == cols, jnp.inf, d2)
        bits = jax.lax.bitcast_convert_type(d2, jnp.int32)
        key = bits ^ ((bits >> 31) & jnp.int32(0x7FFFFFFF))
        kp = (key & jnp.int32(~0x7FF)) | cols
        q0, q1, q2, q3 = (kp[:, :512], kp[:, 512:1024],
                          kp[:, 1024:1536], kp[:, 1536:])
        a0, a1 = jnp.minimum(q0, q1), jnp.maximum(q0, q1)
        b0, b1 = jnp.minimum(q2, q3), jnp.maximum(q2, q3)
        s0, s3 = jnp.minimum(a0, b0), jnp.maximum(a1, b1)
        mid1, mid2 = jnp.maximum(a0, b0), jnp.minimum(a1, b1)
        s1, s2 = jnp.minimum(mid1, mid2), jnp.maximum(mid1, mid2)
        pos = jax.lax.broadcasted_iota(jnp.int32, (SLAB, 512), 1)
        picks = []
        for _ in range(KNN):
            m = jnp.min(s0, axis=1, keepdims=True)
            picks.append(m & jnp.int32(0x7FF))
            cond = pos == (m & jnp.int32(0x1FF))
            s0 = jnp.where(cond, s1, s0)
            s1 = jnp.where(cond, s2, s1)
            s2 = jnp.where(cond, s3, s2)
            s3 = jnp.where(cond, maxi, s3)
        nn_slabs.append(jnp.concatenate(picks, axis=1))
    nn = jnp.concatenate(nn_slabs, axis=0)
    nn_ref[0] = nn
    r16 = rb * RB + jax.lax.broadcasted_iota(jnp.int32, (RB, KNN), 0)
    dup = jnp.zeros((RB, KNN), dtype=jnp.bool_)
    for off in OFFS:
        tgt = r16 + off
        dup = dup | ((nn == tgt) & (tgt >= 0) & (tgt < N))
    wk_ref[0] = 1.0 - dup.astype(jnp.float32)
    h = h_ref[0]
    a_ref[0] = jax.lax.dot_general(h, w1a_ref[...], (((1,), (0,)), ((), ())),
                                   preferred_element_type=jnp.float32) + b1_ref[...]
    bb_ref[0] = jax.lax.dot_general(h, w1b_ref[...], (((1,), (0,)), ((), ())),
                                    preferred_element_type=jnp.float32)


def _run_knn(xpad, xt, hidden, w1a, w1b, b1, interpret=False):
    grid = (B, N // RB)
    return pl.pallas_call(
        _knn_body,
        grid=grid,
        in_specs=[
            pl.BlockSpec((1, RB, 16), lambda b, r: (b, r, 0)),
            pl.BlockSpec((1, 16, N), lambda b, r: (b, 0, 0)),
            pl.BlockSpec((1, RB, HID), lambda b, r: (b, r, 0)),
            pl.BlockSpec((HID, HID), lambda b, r: (0, 0)),
            pl.BlockSpec((HID, HID), lambda b, r: (0, 0)),
            pl.BlockSpec((1, HID), lambda b, r: (0, 0)),
        ],
        out_specs=[
            pl.BlockSpec((1, RB, KNN), lambda b, r: (b, r, 0)),
            pl.BlockSpec((1, RB, KNN), lambda b, r: (b, r, 0)),
            pl.BlockSpec((1, RB, HID), lambda b, r: (b, r, 0)),
            pl.BlockSpec((1, RB, HID), lambda b, r: (b, r, 0)),
        ],
        out_shape=[
            jax.ShapeDtypeStruct((B, N, KNN), jnp.int32),
            jax.ShapeDtypeStruct((B, N, KNN), jnp.float32),
            jax.ShapeDtypeStruct((B, N, HID), jnp.float32),
            jax.ShapeDtypeStruct((B, N, HID), jnp.float32),
        ],
        interpret=interpret,
    )(xpad, xt, hidden, w1a, w1b, b1)


def _egnn_body(h_ref, xpad_ref, a_ref, wk_ref, bg_ref, xg_ref,
               w2_ref, b2_ref, cw1_ref, cb1_ref, cw2_ref, cb2_ref,
               nw1a_ref, nw1b_ref, nb1_ref, nw2_ref, nb2_ref,
               lng_ref, lnb_ref, wlast_ref,
               ho_ref, xo_ref, aggm, aggd, deg):
    k = pl.program_id(1)

    @pl.when(k == 0)
    def _():
        aggm[...] = jnp.zeros_like(aggm)
        aggd[...] = jnp.zeros_like(aggd)
        deg[...] = jnp.zeros_like(deg)

    xi = xpad_ref[0]
    xg = xg_ref[0]
    lane16 = jax.lax.broadcasted_iota(jnp.int32, (N, 16), 1)
    rel = jnp.where(lane16 < 3, xi - xg, 0.0)
    dist2 = jnp.sum(rel * rel, axis=1, keepdims=True)

    icol = jax.lax.broadcasted_iota(jnp.int32, (N, 1), 0)
    off = jnp.where(k == 0, -2, jnp.where(k == 1, -1, jnp.where(k == 2, 1, 2)))
    tgt = icol + off
    wseq = ((tgt >= 0) & (tgt < N)).astype(jnp.float32)
    lane = jax.lax.broadcasted_iota(jnp.int32, (N, KNN), 1)
    wknn = jnp.sum(wk_ref[0] * (lane == (k - NSEQ)).astype(jnp.float32),
                   axis=1, keepdims=True)
    w = jnp.where(k < NSEQ, wseq, wknn)

    pre1 = a_ref[0] + bg_ref[0] + dist2 * wlast_ref[...]
    t = _silu(pre1)
    msg = _silu(jax.lax.dot_general(t.astype(jnp.bfloat16), w2_ref[...],
                                    (((1,), (0,)), ((), ())),
                                    preferred_element_type=jnp.float32) + b2_ref[...])
    c1 = _silu(jax.lax.dot_general(msg.astype(jnp.bfloat16), cw1_ref[...],
                                   (((1,), (0,)), ((), ())),
                                   preferred_element_type=jnp.float32) + cb1_ref[...])
    coef = jnp.tanh(jnp.sum(c1 * cw2_ref[...], axis=1, keepdims=True) + cb2_ref[0, 0])

    aggm[...] += msg * w
    aggd[...] += rel * (coef * w)
    deg[...] += w

    @pl.when(k == NSLOT - 1)
    def _():
        h = h_ref[0]
        hm1 = (jax.lax.dot_general(h.astype(jnp.bfloat16), nw1a_ref[...],
                                   (((1,), (0,)), ((), ())),
                                   preferred_element_type=jnp.float32)
               + jax.lax.dot_general(aggm[...].astype(jnp.bfloat16), nw1b_ref[...],
                                     (((1,), (0,)), ((), ())),
                                     preferred_element_type=jnp.float32)
               + nb1_ref[...])
        hn = h + jax.lax.dot_general(_silu(hm1).astype(jnp.bfloat16), nw2_ref[...],
                                     (((1,), (0,)), ((), ())),
                                     preferred_element_type=jnp.float32) + nb2_ref[...]
        mu = jnp.mean(hn, axis=1, keepdims=True)
        var = jnp.mean((hn - mu) ** 2, axis=1, keepdims=True)
        ho_ref[0] = (hn - mu) / jnp.sqrt(var + 1e-5) * lng_ref[...] + lnb_ref[...]
        xo_ref[0] = xi + STEP * aggd[...] / jnp.maximum(deg[...], 1.0)


def _run_egnn(hidden, xpad, A, wk, bg, xg, p, interpret=False):
    grid = (B, NSLOT)
    cvec = lambda v: v.reshape(1, -1)
    bf = jnp.bfloat16
    w2, b2 = p['edge_w2'].astype(bf), cvec(p['edge_b2'])
    cw1, cb1 = p['coord_w1'].astype(bf), cvec(p['coord_b1'])
    cw2 = p['coord_w2'].reshape(1, HID)
    cb2 = p['coord_b2'].reshape(1, 1)
    nw1a = p['node_w1'][:HID].astype(bf)
    nw1b = p['node_w1'][HID:].astype(bf)
    nb1 = cvec(p['node_b1'])
    nw2, nb2 = p['node_w2'].astype(bf), cvec(p['node_b2'])
    lng, lnb = cvec(p['ln_g']), cvec(p['ln_b'])
    wlast = p['edge_w1'][2 * HID].reshape(1, HID)

    full = lambda shp: pl.BlockSpec(shp, lambda b, k: tuple(0 for _ in shp))
    perb = lambda shp: pl.BlockSpec((1,) + shp, lambda b, k: (b, 0, 0))
    perk = lambda shp: pl.BlockSpec((1,) + shp, lambda b, k: (k * B + b, 0, 0))

    return pl.pallas_call(
        _egnn_body,
        grid=grid,
        in_specs=[
            perb((N, HID)),
            perb((N, 16)),
            perb((N, HID)),
            perb((N, KNN)),
            perk((N, HID)),
            perk((N, 16)),
            full((HID, HID)), full((1, HID)),
            full((HID, HID)), full((1, HID)),
            full((1, HID)), full((1, 1)),
            full((HID, HID)), full((HID, HID)), full((1, HID)),
            full((HID, HID)), full((1, HID)),
            full((1, HID)), full((1, HID)),
            full((1, HID)),
        ],
        out_specs=[
            perb((N, HID)),
            perb((N, 16)),
        ],
        out_shape=[
            jax.ShapeDtypeStruct((B, N, HID), jnp.float32),
            jax.ShapeDtypeStruct((B, N, 16), jnp.float32),
        ],
        scratch_shapes=[
            pltpu.VMEM((N, HID), jnp.float32),
            pltpu.VMEM((N, 16), jnp.float32),
            pltpu.VMEM((N, 1), jnp.float32),
        ],
        interpret=interpret,
    )(hidden, xpad, A, wk, bg, xg,
      w2, b2, cw1, cb1, cw2, cb2, nw1a, nw1b, nb1, nw2, nb2, lng, lnb, wlast)


_NC, _NS = 2, 16
_NW = _NC * _NS
_E = NSLOT * B * N
_PW = _E // _NW
_CH = 128
_NCHUNK = _PW // _CH


def _gather_sc_body(btab_hbm, xflat_hbm, idx_hbm, bg_hbm, xg_hbm,
                    idx_v, xvm, b0, b1, x0, x1, s0, s1):
    wid = lax.axis_index("s") * _NC + lax.axis_index("c")
    base = wid * _PW
    pltpu.sync_copy(idx_hbm.at[pl.ds(base, _PW)], idx_v)
    pltpu.sync_copy(xflat_hbm, xvm)
    bufs_b, bufs_x, sems = (b0, b1), (x0, x1), (s0, s1)
    lane = jax.lax.iota(jnp.int32, 16)

    def start(j, slot):
        idxs = idx_v.at[pl.ds(j * _CH, _CH)]
        pltpu.async_copy(btab_hbm.at[idxs], bufs_b[slot], sems[slot])

    def gather_x(j, slot):
        xb = bufs_x[slot]
        for g in range(_CH // 16):
            rows = idx_v[pl.ds(j * _CH + g * 16, 16)]
            out_base = (g * 16 + lane) * 16
            for c in range(3):
                v = plsc.load_gather(xvm, [rows * 4 + c])
                plsc.store_scatter(xb, [out_base + c], v)

    def wait(slot):
        pltpu.make_async_copy(btab_hbm.at[pl.ds(0, _CH)], bufs_b[slot], sems[slot]).wait()

    def write(j, slot):
        pltpu.sync_copy(bufs_b[slot], bg_hbm.at[pl.ds(base + j * _CH, _CH)])
        pltpu.sync_copy(bufs_x[slot], xg_hbm.at[pl.ds((base + j * _CH) * 16, _CH * 16)])

    start(0, 0)

    @pl.loop(0, _NCHUNK, step=2)
    def _chunks(jj):
        for t in range(2):
            j = jj + t

            @pl.when(j + 1 < _NCHUNK)
            def _():
                start(j + 1, 1 - t)

            gather_x(j, t)
            wait(t)
            write(j, t)


def _run_gather(btab, xflat, gidx):
    mesh = plsc.VectorSubcoreMesh(core_axis_name="c", subcore_axis_name="s")
    f = pl.kernel(
        _gather_sc_body,
        out_type=[
            jax.ShapeDtypeStruct((_E, HID), jnp.float32),
            jax.ShapeDtypeStruct((_E * 16,), jnp.float32),
        ],
        mesh=mesh,
        scratch_types=[
            pltpu.VMEM((_PW,), jnp.int32),
            pltpu.VMEM((B * N * 4,), jnp.float32),
            pltpu.VMEM((_CH, HID), jnp.float32),
            pltpu.VMEM((_CH, HID), jnp.float32),
            pltpu.VMEM((_CH * 16,), jnp.float32),
            pltpu.VMEM((_CH * 16,), jnp.float32),
            pltpu.SemaphoreType.DMA,
            pltpu.SemaphoreType.DMA,
        ],
        compiler_params=pltpu.CompilerParams(needs_layout_passes=False),
    )
    return f(btab, xflat, gidx)


def _assemble(hidden, coords, params, interpret=False):
    xpad = jnp.pad(coords, ((0, 0), (0, 0), (0, 13)))
    xt = jnp.transpose(xpad, (0, 2, 1))
    w1a = params['edge_w1'][:HID]
    w1b = params['edge_w1'][HID:2 * HID]
    b1 = params['edge_b1'].reshape(1, HID)

    nn, wk, A, Btab = _run_knn(xpad, xt, hidden, w1a, w1b, b1, interpret)

    idx = jnp.arange(N, dtype=jnp.int32)
    seq = jnp.stack([jnp.clip(idx + o, 0, N - 1) for o in OFFS], axis=1)
    seq = jnp.broadcast_to(seq[None], (B, N, NSEQ))
    nbr = jnp.concatenate([seq, nn], axis=2)
    gidx = nbr + (jnp.arange(B, dtype=jnp.int32) * N)[:, None, None]
    gidx = jnp.transpose(gidx, (2, 0, 1)).reshape(-1)

    btab = Btab.reshape(B * N, HID)
    if interpret:
        xtab = xpad.reshape(B * N, 16)
        bg = jnp.take(btab, gidx, axis=0)
        xg = jnp.take(xtab, gidx, axis=0).reshape(-1)
    else:
        xflat = jnp.pad(coords, ((0, 0), (0, 0), (0, 1))).reshape(-1)
        bg, xg = _run_gather(btab, xflat, gidx)
    bg = bg.reshape(NSLOT * B, N, HID)
    xg = xg.reshape(NSLOT * B, N, 16)

    ho, xo = _run_egnn(hidden, xpad, A, wk, bg, xg, params, interpret)
    return ho, xo[:, :, :3]


def kernel(hidden, coords, mask, params):
    ho, xo = _assemble(hidden, coords, params)
    return (ho, xo)


# speedup vs baseline: 1.0569x; 1.0011x over previous
import functools

import jax
import jax.numpy as jnp
from jax import lax
from jax.experimental import pallas as pl
from jax.experimental.pallas import tpu as pltpu
from jax.experimental.pallas import tpu_sc as plsc

HID = 128
KNN = 16
NSEQ = 4
NSLOT = NSEQ + KNN
STEP = 0.1
N = 2048
B = 4
RB = 256
OFFS = (-2, -1, 1, 2)


def _silu(v):
    return v * jax.nn.sigmoid(v)


def _knn_body(xpad_ref, xt_ref, h_ref, w1a_ref, w1b_ref, b1_ref,
              nn_ref, wk_ref, a_ref, bb_ref):
    xr = xpad_ref[0]
    xt = xt_ref[0]
    sqj = jnp.sum(xt * xt, axis=0, keepdims=True)
    rb = pl.program_id(1)
    maxi = jnp.int32(0x7FFFFFFF)
    SLAB = 16
    nn_slabs = []
    for sl in range(RB // SLAB):
        xs = xr[sl * SLAB:(sl + 1) * SLAB]
        mm = jax.lax.dot_general(xs, xt, (((1,), (0,)), ((), ())),
                                 preferred_element_type=jnp.float32)
        sqi = jnp.sum(xs * xs, axis=1, keepdims=True)
        d2 = sqi + sqj - 2.0 * mm
        row0 = rb * RB + sl * SLAB
        rows = row0 + jax.lax.broadcasted_iota(jnp.int32, (SLAB, N), 0)
        cols = jax.lax.broadcasted_iota(jnp.int32, (SLAB, N), 1)
        d2 = jnp.where(rows == cols, jnp.inf, d2)
        bits = jax.lax.bitcast_convert_type(d2, jnp.int32)
        key = bits ^ ((bits >> 31) & jnp.int32(0x7FFFFFFF))
        kp = (key & jnp.int32(~0x7FF)) | cols
        q0, q1, q2, q3 = (kp[:, :512], kp[:, 512:1024],
                          kp[:, 1024:1536], kp[:, 1536:])
        a0, a1 = jnp.minimum(q0, q1), jnp.maximum(q0, q1)
        b0, b1 = jnp.minimum(q2, q3), jnp.maximum(q2, q3)
        s0, s3 = jnp.minimum(a0, b0), jnp.maximum(a1, b1)
        mid1, mid2 = jnp.maximum(a0, b0), jnp.minimum(a1, b1)
        s1, s2 = jnp.minimum(mid1, mid2), jnp.maximum(mid1, mid2)
        pos = jax.lax.broadcasted_iota(jnp.int32, (SLAB, 512), 1)
        picks = []
        for _ in range(KNN):
            m = jnp.min(s0, axis=1, keepdims=True)
            picks.append(m & jnp.int32(0x7FF))
            cond = pos == (m & jnp.int32(0x1FF))
            s0 = jnp.where(cond, s1, s0)
            s1 = jnp.where(cond, s2, s1)
            s2 = jnp.where(cond, s3, s2)
            s3 = jnp.where(cond, maxi, s3)
        nn_slabs.append(jnp.concatenate(picks, axis=1))
    nn = jnp.concatenate(nn_slabs, axis=0)
    nn_ref[0] = nn
    r16 = rb * RB + jax.lax.broadcasted_iota(jnp.int32, (RB, KNN), 0)
    dup = jnp.zeros((RB, KNN), dtype=jnp.bool_)
    for off in OFFS:
        tgt = r16 + off
        dup = dup | ((nn == tgt) & (tgt >= 0) & (tgt < N))
    wk_ref[0] = 1.0 - dup.astype(jnp.float32)
    h = h_ref[0]
    a_ref[0] = jax.lax.dot_general(h, w1a_ref[...], (((1,), (0,)), ((), ())),
                                   preferred_element_type=jnp.float32) + b1_ref[...]
    bb_ref[0] = jax.lax.dot_general(h, w1b_ref[...], (((1,), (0,)), ((), ())),
                                    preferred_element_type=jnp.float32)


def _run_knn(xpad, xt, hidden, w1a, w1b, b1):
    grid = (B, N // RB)
    return pl.pallas_call(
        _knn_body,
        grid=grid,
        in_specs=[
            pl.BlockSpec((1, RB, 16), lambda b, r: (b, r, 0)),
            pl.BlockSpec((1, 16, N), lambda b, r: (b, 0, 0)),
            pl.BlockSpec((1, RB, HID), lambda b, r: (b, r, 0)),
            pl.BlockSpec((HID, HID), lambda b, r: (0, 0)),
            pl.BlockSpec((HID, HID), lambda b, r: (0, 0)),
            pl.BlockSpec((1, HID), lambda b, r: (0, 0)),
        ],
        out_specs=[
            pl.BlockSpec((1, RB, KNN), lambda b, r: (b, r, 0)),
            pl.BlockSpec((1, RB, KNN), lambda b, r: (b, r, 0)),
            pl.BlockSpec((1, RB, HID), lambda b, r: (b, r, 0)),
            pl.BlockSpec((1, RB, HID), lambda b, r: (b, r, 0)),
        ],
        out_shape=[
            jax.ShapeDtypeStruct((B, N, KNN), jnp.int32),
            jax.ShapeDtypeStruct((B, N, KNN), jnp.float32),
            jax.ShapeDtypeStruct((B, N, HID), jnp.float32),
            jax.ShapeDtypeStruct((B, N, HID), jnp.float32),
        ],
    )(xpad, xt, hidden, w1a, w1b, b1)


def _egnn_body(h_ref, xpad_ref, a_ref, wk_ref, bg_ref, xg_ref,
               w2_ref, b2_ref, cw1_ref, cb1_ref, cw2_ref, cb2_ref,
               nw1a_ref, nw1b_ref, nb1_ref, nw2_ref, nb2_ref,
               lng_ref, lnb_ref, wlast_ref,
               ho_ref, xo_ref, aggm, aggd, deg):
    k = pl.program_id(1)

    @pl.when(k == 0)
    def _():
        aggm[...] = jnp.zeros_like(aggm)
        aggd[...] = jnp.zeros_like(aggd)
        deg[...] = jnp.zeros_like(deg)

    xi = xpad_ref[0]
    xg = xg_ref[0]
    lane16 = jax.lax.broadcasted_iota(jnp.int32, (N, 16), 1)
    rel = jnp.where(lane16 < 3, xi - xg, 0.0)
    dist2 = jnp.sum(rel * rel, axis=1, keepdims=True)

    icol = jax.lax.broadcasted_iota(jnp.int32, (N, 1), 0)
    off = jnp.where(k == 0, -2, jnp.where(k == 1, -1, jnp.where(k == 2, 1, 2)))
    tgt = icol + off
    wseq = ((tgt >= 0) & (tgt < N)).astype(jnp.float32)
    lane = jax.lax.broadcasted_iota(jnp.int32, (N, KNN), 1)
    wknn = jnp.sum(wk_ref[0] * (lane == (k - NSEQ)).astype(jnp.float32),
                   axis=1, keepdims=True)
    w = jnp.where(k < NSEQ, wseq, wknn)

    pre1 = a_ref[0] + bg_ref[0] + dist2 * wlast_ref[...]
    t = _silu(pre1)
    msg = _silu(jax.lax.dot_general(t.astype(jnp.bfloat16), w2_ref[...],
                                    (((1,), (0,)), ((), ())),
                                    preferred_element_type=jnp.float32) + b2_ref[...])
    c1 = _silu(jax.lax.dot_general(msg.astype(jnp.bfloat16), cw1_ref[...],
                                   (((1,), (0,)), ((), ())),
                                   preferred_element_type=jnp.float32) + cb1_ref[...])
    coef = jnp.tanh(jnp.sum(c1 * cw2_ref[...], axis=1, keepdims=True) + cb2_ref[0, 0])

    aggm[...] += msg * w
    aggd[...] += rel * (coef * w)
    deg[...] += w

    @pl.when(k == NSLOT - 1)
    def _():
        h = h_ref[0]
        hm1 = (jax.lax.dot_general(h.astype(jnp.bfloat16), nw1a_ref[...],
                                   (((1,), (0,)), ((), ())),
                                   preferred_element_type=jnp.float32)
               + jax.lax.dot_general(aggm[...].astype(jnp.bfloat16), nw1b_ref[...],
                                     (((1,), (0,)), ((), ())),
                                     preferred_element_type=jnp.float32)
               + nb1_ref[...])
        hn = h + jax.lax.dot_general(_silu(hm1).astype(jnp.bfloat16), nw2_ref[...],
                                     (((1,), (0,)), ((), ())),
                                     preferred_element_type=jnp.float32) + nb2_ref[...]
        mu = jnp.mean(hn, axis=1, keepdims=True)
        var = jnp.mean((hn - mu) ** 2, axis=1, keepdims=True)
        ho_ref[0] = (hn - mu) / jnp.sqrt(var + 1e-5) * lng_ref[...] + lnb_ref[...]
        xo_ref[0] = xi + STEP * aggd[...] / jnp.maximum(deg[...], 1.0)


def _run_egnn(hidden, xpad, A, wk, bg, xg, p):
    grid = (B, NSLOT)
    cvec = lambda v: v.reshape(1, -1)
    bf = jnp.bfloat16
    w2, b2 = p['edge_w2'].astype(bf), cvec(p['edge_b2'])
    cw1, cb1 = p['coord_w1'].astype(bf), cvec(p['coord_b1'])
    cw2 = p['coord_w2'].reshape(1, HID)
    cb2 = p['coord_b2'].reshape(1, 1)
    nw1a = p['node_w1'][:HID].astype(bf)
    nw1b = p['node_w1'][HID:].astype(bf)
    nb1 = cvec(p['node_b1'])
    nw2, nb2 = p['node_w2'].astype(bf), cvec(p['node_b2'])
    lng, lnb = cvec(p['ln_g']), cvec(p['ln_b'])
    wlast = p['edge_w1'][2 * HID].reshape(1, HID)

    full = lambda shp: pl.BlockSpec(shp, lambda b, k: tuple(0 for _ in shp))
    perb = lambda shp: pl.BlockSpec((1,) + shp, lambda b, k: (b, 0, 0))
    perk = lambda shp: pl.BlockSpec((1,) + shp, lambda b, k: (k * B + b, 0, 0))

    return pl.pallas_call(
        _egnn_body,
        grid=grid,
        in_specs=[
            perb((N, HID)),
            perb((N, 16)),
            perb((N, HID)),
            perb((N, KNN)),
            perk((N, HID)),
            perk((N, 16)),
            full((HID, HID)), full((1, HID)),
            full((HID, HID)), full((1, HID)),
            full((1, HID)), full((1, 1)),
            full((HID, HID)), full((HID, HID)), full((1, HID)),
            full((HID, HID)), full((1, HID)),
            full((1, HID)), full((1, HID)),
            full((1, HID)),
        ],
        out_specs=[
            perb((N, HID)),
            perb((N, 16)),
        ],
        out_shape=[
            jax.ShapeDtypeStruct((B, N, HID), jnp.float32),
            jax.ShapeDtypeStruct((B, N, 16), jnp.float32),
        ],
        scratch_shapes=[
            pltpu.VMEM((N, HID), jnp.float32),
            pltpu.VMEM((N, 16), jnp.float32),
            pltpu.VMEM((N, 1), jnp.float32),
        ],
    )(hidden, xpad, A, wk, bg, xg,
      w2, b2, cw1, cb1, cw2, cb2, nw1a, nw1b, nb1, nw2, nb2, lng, lnb, wlast)


_NC, _NS = 2, 16
_NW = _NC * _NS
_E = NSLOT * B * N
_PW = _E // _NW
_CH = 128
_NCHUNK = _PW // _CH


def _gather_sc_body(btab_hbm, xflat_hbm, idx_hbm, bg_hbm, xg_hbm,
                    idx_v, xvm, b0, b1, x0, x1, s0, s1):
    wid = lax.axis_index("s") * _NC + lax.axis_index("c")
    base = wid * _PW
    pltpu.sync_copy(idx_hbm.at[pl.ds(base, _PW)], idx_v)
    pltpu.sync_copy(xflat_hbm, xvm)
    bufs_b, bufs_x, sems = (b0, b1), (x0, x1), (s0, s1)
    lane = jax.lax.iota(jnp.int32, 16)

    def start(j, slot):
        idxs = idx_v.at[pl.ds(j * _CH, _CH)]
        pltpu.async_copy(btab_hbm.at[idxs], bufs_b[slot], sems[slot])

    def gather_x(j, slot):
        xb = bufs_x[slot]
        for g in range(_CH // 16):
            rows = idx_v[pl.ds(j * _CH + g * 16, 16)]
            out_base = (g * 16 + lane) * 16
            for c in range(3):
                v = plsc.load_gather(xvm, [rows * 4 + c])
                plsc.store_scatter(xb, [out_base + c], v)

    def wait(slot):
        pltpu.make_async_copy(btab_hbm.at[pl.ds(0, _CH)], bufs_b[slot], sems[slot]).wait()

    def write(j, slot):
        pltpu.sync_copy(bufs_b[slot], bg_hbm.at[pl.ds(base + j * _CH, _CH)])
        pltpu.sync_copy(bufs_x[slot], xg_hbm.at[pl.ds((base + j * _CH) * 16, _CH * 16)])

    start(0, 0)

    @pl.loop(0, _NCHUNK, step=2)
    def _chunks(jj):
        for t in range(2):
            j = jj + t

            @pl.when(j + 1 < _NCHUNK)
            def _():
                start(j + 1, 1 - t)

            gather_x(j, t)
            wait(t)
            write(j, t)


def _run_gather(btab, xflat, gidx):
    mesh = plsc.VectorSubcoreMesh(core_axis_name="c", subcore_axis_name="s")
    f = pl.kernel(
        _gather_sc_body,
        out_type=[
            jax.ShapeDtypeStruct((_E, HID), jnp.float32),
            jax.ShapeDtypeStruct((_E * 16,), jnp.float32),
        ],
        mesh=mesh,
        scratch_types=[
            pltpu.VMEM((_PW,), jnp.int32),
            pltpu.VMEM((B * N * 4,), jnp.float32),
            pltpu.VMEM((_CH, HID), jnp.float32),
            pltpu.VMEM((_CH, HID), jnp.float32),
            pltpu.VMEM((_CH * 16,), jnp.float32),
            pltpu.VMEM((_CH * 16,), jnp.float32),
            pltpu.SemaphoreType.DMA,
            pltpu.SemaphoreType.DMA,
        ],
        compiler_params=pltpu.CompilerParams(needs_layout_passes=False),
    )
    return f(btab, xflat, gidx)


def _assemble(hidden, coords, params):
    xpad = jnp.pad(coords, ((0, 0), (0, 0), (0, 13)))
    xt = jnp.transpose(xpad, (0, 2, 1))
    w1a = params['edge_w1'][:HID]
    w1b = params['edge_w1'][HID:2 * HID]
    b1 = params['edge_b1'].reshape(1, HID)

    nn, wk, A, Btab = _run_knn(xpad, xt, hidden, w1a, w1b, b1)

    idx = jnp.arange(N, dtype=jnp.int32)
    seq = jnp.stack([jnp.clip(idx + o, 0, N - 1) for o in OFFS], axis=1)
    seq = jnp.broadcast_to(seq[None], (B, N, NSEQ))
    nbr = jnp.concatenate([seq, nn], axis=2)
    gidx = nbr + (jnp.arange(B, dtype=jnp.int32) * N)[:, None, None]
    gidx = jnp.transpose(gidx, (2, 0, 1)).reshape(-1)

    btab = Btab.reshape(B * N, HID)
    xflat = jnp.pad(coords, ((0, 0), (0, 0), (0, 1))).reshape(-1)
    bg, xg = _run_gather(btab, xflat, gidx)
    bg = bg.reshape(NSLOT * B, N, HID)
    xg = xg.reshape(NSLOT * B, N, 16)

    ho, xo = _run_egnn(hidden, xpad, A, wk, bg, xg, params)
    return ho, xo[:, :, :3]


def kernel(hidden, coords, mask, params):
    ho, xo = _assemble(hidden, coords, params)
    return (ho, xo)
